# baseline (speedup 1.0000x reference)
import functools

import jax
import jax.numpy as jnp
from jax import lax
from jax.experimental import pallas as pl
from jax.experimental.pallas import tpu as pltpu
from jax.experimental.pallas import tpu_sc as plsc

_L = 16
_C = 16
_G = 39


def _emb_call(B, D, Dp, NC, NS):
    NW = NC * NS
    b_per_w = B // NW
    nch = b_per_w // _C
    npair = nch // 2
    mesh = plsc.VectorSubcoreMesh(core_axis_name="c", subcore_axis_name="s")

    @functools.partial(
        pl.kernel,
        mesh=mesh,
        out_type=jax.ShapeDtypeStruct((B * _G, _L), jnp.float32),
        compiler_params=pltpu.CompilerParams(
            use_tc_tiling_on_sc=False, needs_layout_passes=False),
        scratch_types=[
            pltpu.VMEM((b_per_w * _G,), jnp.int32),
            pltpu.VMEM((_C * _G, _L), jnp.float32),
            pltpu.VMEM((_C * _G, _L), jnp.float32),
            pltpu.SemaphoreType.DMA,
            pltpu.SemaphoreType.DMA,
            pltpu.SemaphoreType.DMA,
            pltpu.SemaphoreType.DMA,
        ],
    )
    def emb(idx_hbm, view_hbm, out_hbm,
            idx_v, st0, st1, gsem0, gsem1, ssem0, ssem1):
        wid = lax.axis_index("s") * NC + lax.axis_index("c")
        base = wid * b_per_w
        pltpu.sync_copy(
            idx_hbm.at[pl.ds(base * _G, b_per_w * _G)], idx_v)

        def start_gather(chunk, st, gsem):
            pltpu.async_copy(
                view_hbm.at[idx_v.at[pl.ds(chunk * (_C * _G), _C * _G)]],
                st, gsem)

        def wait_gather(st, gsem):
            pltpu.make_async_copy(
                view_hbm.at[pl.ds(0, _C * _G)], st, gsem).wait()

        def start_store(chunk, st, ssem):
            pltpu.async_copy(
                st, out_hbm.at[pl.ds((base + chunk * _C) * _G, _C * _G)],
                ssem)

        def wait_store(st, ssem):
            pltpu.make_async_copy(
                out_hbm.at[pl.ds(0, _C * _G)], st, ssem).wait()

        start_gather(0, st0, gsem0)

        def pair(t, carry):
            ca = 2 * t
            cb = 2 * t + 1

            @pl.when(t > 0)
            def _():
                wait_store(st1, ssem1)

            start_gather(cb, st1, gsem1)
            wait_gather(st0, gsem0)
            start_store(ca, st0, ssem0)

            @pl.when(t < npair - 1)
            def _():
                wait_store(st0, ssem0)
                start_gather(ca + 2, st0, gsem0)

            wait_gather(st1, gsem1)
            start_store(cb, st1, ssem1)
            return carry

        lax.fori_loop(0, npair, pair, 0)
        wait_store(st0, ssem0)
        wait_store(st1, ssem1)

    return emb


def kernel(input_sentences, embedding_weight):
    S0, S1 = input_sentences.shape
    V, D = embedding_weight.shape
    B = S0 * S1
    pad = (-D) % _L
    Dp = D + pad
    info = plsc.get_sparse_core_info()
    NC, NS = info.num_cores, info.num_subcores
    idx = input_sentences.reshape(B).astype(jnp.int32)
    idx_exp = (idx[:, None] * jnp.int32(_G)
               + jnp.arange(_G, dtype=jnp.int32)[None, :]).reshape(-1)
    view = jnp.pad(embedding_weight, ((0, 0), (0, pad))).reshape(
        V * _G, _L)
    out = _emb_call(B, D, Dp, NC, NS)(idx_exp, view)
    return out.reshape(B, Dp)[:, :D].reshape(S0, S1, D)


# speedup vs baseline: 1.0150x; 1.0150x over previous
import functools

import jax
import jax.numpy as jnp
from jax import lax
from jax.experimental import pallas as pl
from jax.experimental.pallas import tpu as pltpu
from jax.experimental.pallas import tpu_sc as plsc

_L = 16
_C = 80


def _emb_call(B, D, Dp, NC, NS):
    NW = NC * NS
    b_per_w = B // NW
    nch = b_per_w // _C
    npair = nch // 2
    mesh = plsc.VectorSubcoreMesh(core_axis_name="c", subcore_axis_name="s")

    @functools.partial(
        pl.kernel,
        mesh=mesh,
        out_type=jax.ShapeDtypeStruct((B, Dp), jnp.float32),
        compiler_params=pltpu.CompilerParams(
            use_tc_tiling_on_sc=False, needs_layout_passes=False),
        scratch_types=[
            pltpu.VMEM((b_per_w,), jnp.int32),
            pltpu.VMEM((_C, Dp), jnp.float32),
            pltpu.VMEM((_C, Dp), jnp.float32),
            pltpu.SemaphoreType.DMA,
            pltpu.SemaphoreType.DMA,
            pltpu.SemaphoreType.DMA,
            pltpu.SemaphoreType.DMA,
        ],
    )
    def emb(idx_hbm, view_hbm, out_hbm,
            idx_v, st0, st1, gsem0, gsem1, ssem0, ssem1):
        wid = lax.axis_index("s") * NC + lax.axis_index("c")
        base = wid * b_per_w
        pltpu.sync_copy(idx_hbm.at[pl.ds(base, b_per_w)], idx_v)

        def start_gather(chunk, st, gsem):
            pltpu.async_copy(
                view_hbm.at[idx_v.at[pl.ds(chunk * _C, _C)]], st, gsem)

        def wait_gather(st, gsem):
            pltpu.make_async_copy(view_hbm.at[pl.ds(0, _C)], st, gsem).wait()

        def start_store(chunk, st, ssem):
            pltpu.async_copy(
                st, out_hbm.at[pl.ds(base + chunk * _C, _C)], ssem)

        def wait_store(st, ssem):
            pltpu.make_async_copy(
                out_hbm.at[pl.ds(0, _C)], st, ssem).wait()

        start_gather(0, st0, gsem0)

        def pair(t, carry):
            ca = 2 * t
            cb = 2 * t + 1

            @pl.when(t > 0)
            def _():
                wait_store(st1, ssem1)

            start_gather(cb, st1, gsem1)
            wait_gather(st0, gsem0)
            start_store(ca, st0, ssem0)

            @pl.when(t < npair - 1)
            def _():
                wait_store(st0, ssem0)
                start_gather(ca + 2, st0, gsem0)

            wait_gather(st1, gsem1)
            start_store(cb, st1, ssem1)
            return carry

        lax.fori_loop(0, npair, pair, 0)
        wait_store(st0, ssem0)
        wait_store(st1, ssem1)

    return emb


def kernel(input_sentences, embedding_weight):
    S0, S1 = input_sentences.shape
    V, D = embedding_weight.shape
    B = S0 * S1
    pad = (-D) % _L
    Dp = D + pad
    info = plsc.get_sparse_core_info()
    NC, NS = info.num_cores, info.num_subcores
    idx = input_sentences.reshape(B).astype(jnp.int32)
    view = jnp.pad(embedding_weight, ((0, 0), (0, pad)))
    out = _emb_call(B, D, Dp, NC, NS)(idx, view)
    return out[:, :D].reshape(S0, S1, D)
